# baseline (speedup 1.0000x reference)
import functools

import jax
import jax.numpy as jnp
from jax import lax
from jax.experimental import pallas as pl
from jax.experimental.pallas import tpu as pltpu
from jax.experimental.pallas import tpu_sc as plsc

NUM_NODE = 50000
NUM_REL = 512
BATCH = 256
NUM_EDGE = 1600000

ROWS = BATCH + 1
DUMMY_ROW = BATCH
REP_PAD = 520
LANE = 16
MICRO = 128
NC = 2
NS = 16
NW = NC * NS
SCAT_RING = 16

MICROS_PER_TILE = 392
MICROS_PER_MACRO = 28
E_PAD = NW * MICRO * MICROS_PER_TILE


def _make_sc_scatter(num_node, e_pad, micros_per_macro, interpret=False):
  micros_per_tile = e_pad // (NW * MICRO)
  assert micros_per_tile * NW * MICRO == e_pad
  assert micros_per_tile % micros_per_macro == 0
  n_macro = micros_per_tile // micros_per_macro
  assert n_macro % 2 == 0 and n_macro >= 2
  macro = micros_per_macro * MICRO
  groups_per_macro = macro // LANE
  tile_edges = micros_per_tile * MICRO

  mesh = plsc.VectorSubcoreMesh(
      core_axis_name="c", subcore_axis_name="s", num_cores=NC, num_subcores=NS)
  scratch = (
      [pltpu.VMEM((REP_PAD,), jnp.int32)]
      + [pltpu.VMEM((macro,), jnp.int32)] * 4
      + [pltpu.VMEM((MICRO,), jnp.float32)]
      + [pltpu.VMEM((SCAT_RING, MICRO), jnp.int32)]
      + [pltpu.SemaphoreType.DMA] * (4 + SCAT_RING)
  )

  @functools.partial(
      pl.kernel, mesh=mesh, out_type=(), scratch_types=scratch,
      compiler_params=pltpu.CompilerParams(needs_layout_passes=False),
      interpret=interpret)
  def scatter_kernel(et_hbm, er_hbm, rep_hbm, m_hbm, *scr):
    rep_v, et0, et1, er0, er1, ones_v, acc = scr[:7]
    sems = list(scr[7:])
    isems = [(sems[0], sems[2]), (sems[1], sems[3])]
    ssems = sems[4:]
    wid = lax.axis_index("s") * NC + lax.axis_index("c")
    base = wid * tile_edges
    et_bufs = [et0, et1]
    er_bufs = [er0, er1]

    pltpu.sync_copy(rep_hbm, rep_v)
    iota16 = lax.iota(jnp.int32, LANE)
    for k in range(MICRO // LANE):
      ones_v[pl.ds(k * LANE, LANE)] = jnp.full((LANE,), 1.0, jnp.float32)
    for jr in range(SCAT_RING):
      for jc in range(MICRO // LANE):
        acc[jr, pl.ds(jc * LANE, LANE)] = (
            jnp.int32(DUMMY_ROW * num_node + (jr * MICRO // LANE + jc) * LANE)
            + iota16)

    def stage(m, b):
      off = base + m * macro
      pltpu.make_async_copy(
          et_hbm.at[pl.ds(off, macro)], et_bufs[b], isems[b][0]).start()
      pltpu.make_async_copy(
          er_hbm.at[pl.ds(off, macro)], er_bufs[b], isems[b][1]).start()

    def wait_stage(b):
      pltpu.make_async_copy(
          et_hbm.at[pl.ds(0, macro)], et_bufs[b], isems[b][0]).wait()
      pltpu.make_async_copy(
          er_hbm.at[pl.ds(0, macro)], er_bufs[b], isems[b][1]).wait()

    def fire_row(r):
      pltpu.make_async_copy(
          ones_v, m_hbm.at[acc.at[jnp.int32(r)]], ssems[r]).start()

    def wait_row(r):
      pltpu.make_async_copy(
          ones_v, m_hbm.at[acc.at[jnp.int32(r)]], ssems[r]).wait()

    stage(0, 0)
    stage(1, 1)

    def group_step(g, carry, ebuf, rbuf):
      cnt, tf = carry
      o = g * LANE
      t = ebuf[pl.ds(o, LANE)]
      r = rbuf[pl.ds(o, LANE)]
      row = plsc.load_gather(rep_v, [r])
      valid = row != jnp.int32(DUMMY_ROW)
      flat = row * num_node + t
      pc = jnp.max(plsc.all_reduce_population_count(valid))
      rrow = lax.shift_right_logical(cnt, jnp.int32(7)) & jnp.int32(SCAT_RING - 1)
      c = cnt & jnp.int32(MICRO - 1)
      plsc.store_compressed(acc.at[rrow, pl.ds(c, LANE)], flat, mask=valid)
      fire = (c + pc) > jnp.int32(MICRO - LANE)
      for rs in range(SCAT_RING):
        @pl.when(fire & (rrow == jnp.int32(rs)))
        def _():
          fire_row(rs)
          @pl.when(tf >= jnp.int32(SCAT_RING - 1))
          def _():
            wait_row((rs + 1) % SCAT_RING)
      cnt = jnp.where(fire, cnt - c + jnp.int32(MICRO), cnt + pc)
      tf = tf + jnp.where(fire, jnp.int32(1), jnp.int32(0))
      return cnt, tf

    def macro_body(m, b, carry):
      wait_stage(b)
      carry = lax.fori_loop(
          jnp.int32(0), jnp.int32(groups_per_macro),
          lambda g, cr: group_step(g, cr, et_bufs[b], er_bufs[b]), carry)

      @pl.when(m + 2 < n_macro)
      def _():
        stage(m + 2, b)
      return carry

    def outer(i, carry):
      carry = macro_body(2 * i, 0, carry)
      carry = macro_body(2 * i + 1, 1, carry)
      return carry

    cnt, tf = lax.fori_loop(
        jnp.int32(0), jnp.int32(n_macro // 2), outer,
        (jnp.int32(0), jnp.int32(0)))

    rrow = lax.shift_right_logical(cnt, jnp.int32(7)) & jnp.int32(SCAT_RING - 1)
    for rs in range(SCAT_RING):
      @pl.when(rrow == jnp.int32(rs))
      def _():
        fire_row(rs)
    tf = tf + jnp.int32(1)
    for rs in range(SCAT_RING):
      @pl.when((tf > jnp.int32(rs)))
      def _():
        wait_row(rs)

  return scatter_kernel


def _tc_logit(m, rand, repb, num_node, rows, batch, interpret=False):
  sub = 8 if num_node % 8 == 0 else 1
  nn = num_node // sub
  m3 = m.reshape(rows, sub, nn)
  r3 = rand.reshape(batch, sub, nn)
  grid_spec = pltpu.PrefetchScalarGridSpec(
      num_scalar_prefetch=1,
      grid=(batch,),
      in_specs=[
          pl.BlockSpec(
              (1, sub, nn),
              lambda b, repb_ref: (repb_ref[b], jnp.int32(0), jnp.int32(0))),
          pl.BlockSpec(
              (1, sub, nn),
              lambda b, repb_ref: (b, jnp.int32(0), jnp.int32(0))),
      ],
      out_specs=pl.BlockSpec(
          (1, sub, nn),
          lambda b, repb_ref: (b, jnp.int32(0), jnp.int32(0))),
  )

  def body(repb_ref, m_blk, rand_blk, out_blk):
    t = (m_blk[...] * 2.0 + rand_blk[...]) / 3.0
    out_blk[...] = jnp.log((t + 1e-10) / (1.0 - t + 1e-10))

  out = pl.pallas_call(
      body,
      grid_spec=grid_spec,
      out_shape=jax.ShapeDtypeStruct((batch, sub, nn), jnp.float32),
      interpret=interpret,
  )(repb, m3, r3)
  return out.reshape(batch, num_node)


def kernel(h_index, r_index, edge_t, edge_r, rand, dummy_param):
  del h_index
  r32 = r_index.astype(jnp.int32)
  big = jnp.int32(2**30)
  rep = jnp.full((NUM_REL,), big, jnp.int32).at[r32].min(
      jnp.arange(BATCH, dtype=jnp.int32))
  repb = rep[r32]
  rep_row = jnp.where(rep < big, rep, jnp.int32(DUMMY_ROW))
  rep_row = jnp.concatenate(
      [rep_row, jnp.full((REP_PAD - NUM_REL,), DUMMY_ROW, jnp.int32)])

  pad = E_PAD - NUM_EDGE
  et_p = jnp.concatenate(
      [edge_t.astype(jnp.int32),
       jnp.arange(pad, dtype=jnp.int32) % jnp.int32(NUM_NODE)])
  er_p = jnp.concatenate(
      [edge_r.astype(jnp.int32), jnp.full((pad,), NUM_REL, jnp.int32)])

  m_ref = jax.new_ref(jnp.zeros((ROWS * NUM_NODE,), jnp.float32))
  _make_sc_scatter(NUM_NODE, E_PAD, MICROS_PER_MACRO)(et_p, er_p, rep_row, m_ref)
  m = m_ref[...]

  out = _tc_logit(m, rand, repb, NUM_NODE, ROWS, BATCH)
  return out + dummy_param[0] * 0.0


# speedup vs baseline: 1.0571x; 1.0571x over previous
import functools

import jax
import jax.numpy as jnp
from jax import lax
from jax.experimental import pallas as pl
from jax.experimental.pallas import tpu as pltpu
from jax.experimental.pallas import tpu_sc as plsc

NUM_NODE = 50000
NUM_REL = 512
BATCH = 256
NUM_EDGE = 1600000

ROWS = BATCH + 1
DUMMY_ROW = BATCH
REP_PAD = 520
LANE = 16
MICRO = 128
NC = 2
NS = 16
NW = NC * NS
SCAT_RING = 16

MICROS_PER_TILE = 392
MICROS_PER_MACRO = 28
E_PAD = NW * MICRO * MICROS_PER_TILE


def _make_sc_scatter(num_node, e_pad, micros_per_macro, interpret=False):
  micros_per_tile = e_pad // (NW * MICRO)
  assert micros_per_tile * NW * MICRO == e_pad
  assert micros_per_tile % micros_per_macro == 0
  n_macro = micros_per_tile // micros_per_macro
  assert n_macro % 2 == 0 and n_macro >= 2
  macro = micros_per_macro * MICRO
  groups_per_macro = macro // LANE
  tile_edges = micros_per_tile * MICRO

  mesh = plsc.VectorSubcoreMesh(
      core_axis_name="c", subcore_axis_name="s", num_cores=NC, num_subcores=NS)
  scratch = (
      [pltpu.VMEM((REP_PAD,), jnp.int32)]
      + [pltpu.VMEM((macro,), jnp.int32)] * 4
      + [pltpu.VMEM((MICRO,), jnp.float32)]
      + [pltpu.VMEM((SCAT_RING, MICRO), jnp.int32)]
      + [pltpu.SemaphoreType.DMA] * (4 + SCAT_RING)
  )

  @functools.partial(
      pl.kernel, mesh=mesh, out_type=(), scratch_types=scratch,
      compiler_params=pltpu.CompilerParams(needs_layout_passes=False),
      interpret=interpret)
  def scatter_kernel(et_hbm, er_hbm, rep_hbm, m_hbm, *scr):
    rep_v, et0, et1, er0, er1, ones_v, acc = scr[:7]
    sems = list(scr[7:])
    isems = [(sems[0], sems[2]), (sems[1], sems[3])]
    ssems = sems[4:]
    wid = lax.axis_index("s") * NC + lax.axis_index("c")
    base = wid * tile_edges
    et_bufs = [et0, et1]
    er_bufs = [er0, er1]

    pltpu.sync_copy(rep_hbm, rep_v)
    iota16 = lax.iota(jnp.int32, LANE)
    for k in range(MICRO // LANE):
      ones_v[pl.ds(k * LANE, LANE)] = jnp.full((LANE,), 1.0, jnp.float32)
    for jr in range(SCAT_RING):
      for jc in range(MICRO // LANE):
        acc[jr, pl.ds(jc * LANE, LANE)] = (
            jnp.int32(DUMMY_ROW * num_node + (jr * MICRO // LANE + jc) * LANE)
            + iota16)

    def stage(m, b):
      off = base + m * macro
      pltpu.make_async_copy(
          et_hbm.at[pl.ds(off, macro)], et_bufs[b], isems[b][0]).start()
      pltpu.make_async_copy(
          er_hbm.at[pl.ds(off, macro)], er_bufs[b], isems[b][1]).start()

    def wait_stage(b):
      pltpu.make_async_copy(
          et_hbm.at[pl.ds(0, macro)], et_bufs[b], isems[b][0]).wait()
      pltpu.make_async_copy(
          er_hbm.at[pl.ds(0, macro)], er_bufs[b], isems[b][1]).wait()

    def fire_row(r):
      pltpu.make_async_copy(
          ones_v, m_hbm.at[acc.at[jnp.int32(r)]], ssems[r]).start()

    def wait_row(r):
      pltpu.make_async_copy(
          ones_v, m_hbm.at[acc.at[jnp.int32(r)]], ssems[r]).wait()

    stage(0, 0)
    stage(1, 1)

    def group_step(g, carry, ebuf, rbuf):
      cnt, tf = carry
      o = g * LANE
      t = ebuf[pl.ds(o, LANE)]
      r = rbuf[pl.ds(o, LANE)]
      row = plsc.load_gather(rep_v, [r])
      valid = row != jnp.int32(DUMMY_ROW)
      flat = row * num_node + t
      pc = jnp.max(plsc.all_reduce_population_count(valid))
      rrow = lax.shift_right_logical(cnt, jnp.int32(7)) & jnp.int32(SCAT_RING - 1)
      c = cnt & jnp.int32(MICRO - 1)
      plsc.store_compressed(acc.at[rrow, pl.ds(c, LANE)], flat, mask=valid)
      fire = (c + pc) > jnp.int32(MICRO - LANE)
      for rs in range(SCAT_RING):
        @pl.when(fire & (rrow == jnp.int32(rs)))
        def _():
          fire_row(rs)
          @pl.when(tf >= jnp.int32(SCAT_RING - 1))
          def _():
            wait_row((rs + 1) % SCAT_RING)
      cnt = jnp.where(fire, cnt - c + jnp.int32(MICRO), cnt + pc)
      tf = tf + jnp.where(fire, jnp.int32(1), jnp.int32(0))
      return cnt, tf

    def macro_body(m, b, carry):
      wait_stage(b)
      carry = lax.fori_loop(
          jnp.int32(0), jnp.int32(groups_per_macro),
          lambda g, cr: group_step(g, cr, et_bufs[b], er_bufs[b]), carry)

      @pl.when(m + 2 < n_macro)
      def _():
        stage(m + 2, b)
      return carry

    def outer(i, carry):
      carry = macro_body(2 * i, 0, carry)
      carry = macro_body(2 * i + 1, 1, carry)
      return carry

    cnt, tf = lax.fori_loop(
        jnp.int32(0), jnp.int32(n_macro // 2), outer,
        (jnp.int32(0), jnp.int32(0)))

    rrow = lax.shift_right_logical(cnt, jnp.int32(7)) & jnp.int32(SCAT_RING - 1)
    for rs in range(SCAT_RING):
      @pl.when(rrow == jnp.int32(rs))
      def _():
        fire_row(rs)
    tf = tf + jnp.int32(1)
    for rs in range(SCAT_RING):
      @pl.when((tf > jnp.int32(rs)))
      def _():
        wait_row(rs)

  return scatter_kernel


def _tc_logit(m, rand, repb, num_node, rows, batch, interpret=False):
  m3 = m.reshape(rows, 1, num_node)
  r3 = rand.reshape(batch, 1, num_node)
  grid_spec = pltpu.PrefetchScalarGridSpec(
      num_scalar_prefetch=1,
      grid=(batch // 2,),
      in_specs=[
          pl.BlockSpec(
              (1, 1, num_node),
              lambda b, repb_ref: (repb_ref[2 * b], jnp.int32(0), jnp.int32(0))),
          pl.BlockSpec(
              (1, 1, num_node),
              lambda b, repb_ref: (repb_ref[2 * b + 1], jnp.int32(0),
                                   jnp.int32(0))),
          pl.BlockSpec(
              (2, 1, num_node),
              lambda b, repb_ref: (b, jnp.int32(0), jnp.int32(0))),
      ],
      out_specs=pl.BlockSpec(
          (2, 1, num_node),
          lambda b, repb_ref: (b, jnp.int32(0), jnp.int32(0))),
  )

  def body(repb_ref, m0_blk, m1_blk, rand_blk, out_blk):
    x = jnp.concatenate([m0_blk[...], m1_blk[...]], axis=0)
    t = (x * 2.0 + rand_blk[...]) / 3.0
    out_blk[...] = jnp.log((t + 1e-10) / (1.0 - t + 1e-10))

  out = pl.pallas_call(
      body,
      grid_spec=grid_spec,
      out_shape=jax.ShapeDtypeStruct((batch, 1, num_node), jnp.float32),
      interpret=interpret,
  )(repb, m3, m3, r3)
  return out.reshape(batch, num_node)


def kernel(h_index, r_index, edge_t, edge_r, rand, dummy_param):
  del h_index
  r32 = r_index.astype(jnp.int32)
  big = jnp.int32(2**30)
  rep = jnp.full((NUM_REL,), big, jnp.int32).at[r32].min(
      jnp.arange(BATCH, dtype=jnp.int32))
  repb = rep[r32]
  rep_row = jnp.where(rep < big, rep, jnp.int32(DUMMY_ROW))
  rep_row = jnp.concatenate(
      [rep_row, jnp.full((REP_PAD - NUM_REL,), DUMMY_ROW, jnp.int32)])

  pad = E_PAD - NUM_EDGE
  et_p = jnp.concatenate(
      [edge_t.astype(jnp.int32),
       jnp.arange(pad, dtype=jnp.int32) % jnp.int32(NUM_NODE)])
  er_p = jnp.concatenate(
      [edge_r.astype(jnp.int32), jnp.full((pad,), NUM_REL, jnp.int32)])

  m_ref = jax.new_ref(jnp.zeros((ROWS * NUM_NODE,), jnp.float32))
  _make_sc_scatter(NUM_NODE, E_PAD, MICROS_PER_MACRO)(et_p, er_p, rep_row, m_ref)
  m = m_ref[...]

  out = _tc_logit(m, rand, repb, NUM_NODE, ROWS, BATCH)
  return out + dummy_param[0] * 0.0


# speedup vs baseline: 1.0741x; 1.0161x over previous
import functools

import jax
import jax.numpy as jnp
from jax import lax
from jax.experimental import pallas as pl
from jax.experimental.pallas import tpu as pltpu
from jax.experimental.pallas import tpu_sc as plsc

NUM_NODE = 50000
NUM_REL = 512
BATCH = 256
NUM_EDGE = 1600000

ROWS = BATCH + 1
DUMMY_ROW = BATCH
REP_PAD = 520
LANE = 16
MICRO = 128
NC = 2
NS = 16
NW = NC * NS
SCAT_RING = 16

MICROS_PER_TILE = 392
MICROS_PER_MACRO = 28
E_PAD = NW * MICRO * MICROS_PER_TILE


def _make_sc_scatter(num_node, e_pad, micros_per_macro, interpret=False):
  micros_per_tile = e_pad // (NW * MICRO)
  assert micros_per_tile * NW * MICRO == e_pad
  assert micros_per_tile % micros_per_macro == 0
  n_macro = micros_per_tile // micros_per_macro
  assert n_macro % 2 == 0 and n_macro >= 2
  macro = micros_per_macro * MICRO
  groups_per_macro = macro // LANE
  tile_edges = micros_per_tile * MICRO

  mesh = plsc.VectorSubcoreMesh(
      core_axis_name="c", subcore_axis_name="s", num_cores=NC, num_subcores=NS)
  scratch = (
      [pltpu.VMEM((REP_PAD,), jnp.int32)]
      + [pltpu.VMEM((macro,), jnp.int32)] * 4
      + [pltpu.VMEM((MICRO,), jnp.float32)]
      + [pltpu.VMEM((SCAT_RING, MICRO), jnp.int32)]
      + [pltpu.SemaphoreType.DMA] * (4 + SCAT_RING)
  )

  @functools.partial(
      pl.kernel, mesh=mesh, out_type=(), scratch_types=scratch,
      compiler_params=pltpu.CompilerParams(needs_layout_passes=False),
      interpret=interpret)
  def scatter_kernel(et_hbm, er_hbm, rep_hbm, m_hbm, *scr):
    rep_v, et0, et1, er0, er1, ones_v, acc = scr[:7]
    sems = list(scr[7:])
    isems = [(sems[0], sems[2]), (sems[1], sems[3])]
    ssems = sems[4:]
    wid = lax.axis_index("s") * NC + lax.axis_index("c")
    base = wid * tile_edges
    et_bufs = [et0, et1]
    er_bufs = [er0, er1]

    pltpu.sync_copy(rep_hbm, rep_v)
    iota16 = lax.iota(jnp.int32, LANE)
    for k in range(MICRO // LANE):
      ones_v[pl.ds(k * LANE, LANE)] = jnp.full((LANE,), 1.0, jnp.float32)
    for jr in range(SCAT_RING):
      for jc in range(MICRO // LANE):
        acc[jr, pl.ds(jc * LANE, LANE)] = (
            jnp.int32(DUMMY_ROW * num_node + (jr * MICRO // LANE + jc) * LANE)
            + iota16)

    def stage(m, b):
      off = base + m * macro
      pltpu.make_async_copy(
          et_hbm.at[pl.ds(off, macro)], et_bufs[b], isems[b][0]).start()
      pltpu.make_async_copy(
          er_hbm.at[pl.ds(off, macro)], er_bufs[b], isems[b][1]).start()

    def wait_stage(b):
      pltpu.make_async_copy(
          et_hbm.at[pl.ds(0, macro)], et_bufs[b], isems[b][0]).wait()
      pltpu.make_async_copy(
          er_hbm.at[pl.ds(0, macro)], er_bufs[b], isems[b][1]).wait()

    def fire_row(r):
      pltpu.make_async_copy(
          ones_v, m_hbm.at[acc.at[jnp.int32(r)]], ssems[r]).start()

    def wait_row(r):
      pltpu.make_async_copy(
          ones_v, m_hbm.at[acc.at[jnp.int32(r)]], ssems[r]).wait()

    stage(0, 0)
    stage(1, 1)

    def group_step(g, carry, ebuf, rbuf):
      cnt, tf = carry
      o = g * LANE
      t = ebuf[pl.ds(o, LANE)]
      r = rbuf[pl.ds(o, LANE)]
      row = plsc.load_gather(rep_v, [r])
      valid = row != jnp.int32(DUMMY_ROW)
      flat = row * num_node + t
      pc = jnp.max(plsc.all_reduce_population_count(valid))
      rrow = lax.shift_right_logical(cnt, jnp.int32(7)) & jnp.int32(SCAT_RING - 1)
      c = cnt & jnp.int32(MICRO - 1)
      plsc.store_compressed(acc.at[rrow, pl.ds(c, LANE)], flat, mask=valid)
      fire = (c + pc) > jnp.int32(MICRO - LANE)
      for rs in range(SCAT_RING):
        @pl.when(fire & (rrow == jnp.int32(rs)))
        def _():
          fire_row(rs)
          @pl.when(tf >= jnp.int32(SCAT_RING - 1))
          def _():
            wait_row((rs + 1) % SCAT_RING)
      cnt = jnp.where(fire, cnt - c + jnp.int32(MICRO), cnt + pc)
      tf = tf + jnp.where(fire, jnp.int32(1), jnp.int32(0))
      return cnt, tf

    def macro_body(m, b, carry):
      wait_stage(b)
      carry = lax.fori_loop(
          jnp.int32(0), jnp.int32(groups_per_macro),
          lambda g, cr: group_step(g, cr, et_bufs[b], er_bufs[b]), carry)

      @pl.when(m + 2 < n_macro)
      def _():
        stage(m + 2, b)
      return carry

    def outer(i, carry):
      carry = macro_body(2 * i, 0, carry)
      carry = macro_body(2 * i + 1, 1, carry)
      return carry

    cnt, tf = lax.fori_loop(
        jnp.int32(0), jnp.int32(n_macro // 2), outer,
        (jnp.int32(0), jnp.int32(0)))

    rrow = lax.shift_right_logical(cnt, jnp.int32(7)) & jnp.int32(SCAT_RING - 1)
    for rs in range(SCAT_RING):
      @pl.when(rrow == jnp.int32(rs))
      def _():
        fire_row(rs)
    tf = tf + jnp.int32(1)
    for rs in range(SCAT_RING):
      @pl.when((tf > jnp.int32(rs)))
      def _():
        wait_row(rs)

  return scatter_kernel


def _tc_logit(m, rand, repb, num_node, rows, batch, interpret=False):
  m3 = m.reshape(rows, 1, num_node)
  r3 = rand.reshape(batch, 1, num_node)
  grid_spec = pltpu.PrefetchScalarGridSpec(
      num_scalar_prefetch=1,
      grid=(batch // 4,),
      in_specs=[
          pl.BlockSpec(
              (1, 1, num_node),
              lambda b, repb_ref, j=j: (repb_ref[4 * b + j], jnp.int32(0),
                                        jnp.int32(0)))
          for j in range(4)
      ] + [
          pl.BlockSpec(
              (4, 1, num_node),
              lambda b, repb_ref: (b, jnp.int32(0), jnp.int32(0))),
      ],
      out_specs=pl.BlockSpec(
          (4, 1, num_node),
          lambda b, repb_ref: (b, jnp.int32(0), jnp.int32(0))),
  )

  def body(repb_ref, m0_blk, m1_blk, m2_blk, m3_blk, rand_blk, out_blk):
    x = jnp.concatenate(
        [m0_blk[...], m1_blk[...], m2_blk[...], m3_blk[...]], axis=0)
    t = (x * 2.0 + rand_blk[...]) / 3.0
    out_blk[...] = jnp.log((t + 1e-10) / (1.0 - t + 1e-10))

  out = pl.pallas_call(
      body,
      grid_spec=grid_spec,
      out_shape=jax.ShapeDtypeStruct((batch, 1, num_node), jnp.float32),
      interpret=interpret,
  )(repb, m3, m3, m3, m3, r3)
  return out.reshape(batch, num_node)


def kernel(h_index, r_index, edge_t, edge_r, rand, dummy_param):
  del h_index
  r32 = r_index.astype(jnp.int32)
  big = jnp.int32(2**30)
  rep = jnp.full((NUM_REL,), big, jnp.int32).at[r32].min(
      jnp.arange(BATCH, dtype=jnp.int32))
  repb = rep[r32]
  rep_row = jnp.where(rep < big, rep, jnp.int32(DUMMY_ROW))
  rep_row = jnp.concatenate(
      [rep_row, jnp.full((REP_PAD - NUM_REL,), DUMMY_ROW, jnp.int32)])

  pad = E_PAD - NUM_EDGE
  et_p = jnp.concatenate(
      [edge_t.astype(jnp.int32),
       jnp.arange(pad, dtype=jnp.int32) % jnp.int32(NUM_NODE)])
  er_p = jnp.concatenate(
      [edge_r.astype(jnp.int32), jnp.full((pad,), NUM_REL, jnp.int32)])

  m_ref = jax.new_ref(jnp.zeros((ROWS * NUM_NODE,), jnp.float32))
  _make_sc_scatter(NUM_NODE, E_PAD, MICROS_PER_MACRO)(et_p, er_p, rep_row, m_ref)
  m = m_ref[...]

  out = _tc_logit(m, rand, repb, NUM_NODE, ROWS, BATCH)
  return out + dummy_param[0] * 0.0
